# baseline (speedup 1.0000x reference)
import functools

import jax
import jax.numpy as jnp
from jax import lax
from jax.experimental import pallas as pl
from jax.experimental.pallas import tpu as pltpu
from jax.experimental.pallas import tpu_sc as plsc

DIM = 256
HALF = 128
NN = 10000
NE = 160000
NSUB = 16
EB = 128
NBLK = 80
CH = 4
NBLK2 = 84
EPAD = NSUB * NBLK * EB
NPAD = 10112
TRASH = NN
ZR = NPAD // NSUB
RB = 2000


def _sc_body(xe0, xe1, src_b, dst_b, attr_b, w2, zrows,
             out0, out1,
             src_st, dst_st, attr_st, w_v, gbuf, sem_g, aggr_sh):
    c = lax.axis_index("c")
    s = lax.axis_index("s")

    pltpu.sync_copy(src_b.at[s], src_st)
    pltpu.sync_copy(dst_b.at[s], dst_st)
    pltpu.sync_copy(attr_b.at[s], attr_st)
    pltpu.sync_copy(w2, w_v)

    pltpu.sync_copy(zrows, aggr_sh.at[pl.ds(s * ZR, ZR)])
    plsc.subcore_barrier()

    wv = [w_v[c, pl.ds(16 * f, 16)] for f in range(8)]

    def block_body(blk, carry):
        @pl.when(c == 0)
        def _():
            pltpu.async_copy(xe0.at[src_st.at[blk]], gbuf, sem_g).wait()

        @pl.when(c == 1)
        def _():
            pltpu.async_copy(xe1.at[src_st.at[blk]], gbuf, sem_g).wait()

        def grp_body(eg, carry2):
            a16 = attr_st[blk, pl.ds(eg * 16, 16)]
            for j in range(16):
                a = a16[j]
                e = eg * 16 + j
                for f in range(8):
                    v = gbuf[e, pl.ds(16 * f, 16)]
                    gbuf[e, pl.ds(16 * f, 16)] = jnp.maximum(v + a * wv[f], 0.0)
            return carry2

        lax.fori_loop(0, EB // 16, grp_body, 0)

        pltpu.sync_copy(gbuf, aggr_sh.at[dst_st.at[blk]], add=True)
        return carry

    lax.fori_loop(0, NBLK, block_body, 0)
    plsc.subcore_barrier()

    @pl.when(c == 0)
    def _():
        pltpu.sync_copy(aggr_sh.at[pl.ds(s * ZR, ZR)], out0.at[pl.ds(s * ZR, ZR)])

    @pl.when(c == 1)
    def _():
        pltpu.sync_copy(aggr_sh.at[pl.ds(s * ZR, ZR)], out1.at[pl.ds(s * ZR, ZR)])


_sc_aggregate = pl.kernel(
    _sc_body,
    out_type=(jax.ShapeDtypeStruct((NPAD, HALF), jnp.float32),
              jax.ShapeDtypeStruct((NPAD, HALF), jnp.float32)),
    mesh=plsc.VectorSubcoreMesh(core_axis_name="c", subcore_axis_name="s"),
    scratch_types=[
        pltpu.VMEM((NBLK, EB), jnp.int32),
        pltpu.VMEM((NBLK, EB), jnp.int32),
        pltpu.VMEM((NBLK, EB), jnp.float32),
        pltpu.VMEM((2, HALF), jnp.float32),
        pltpu.VMEM((EB, HALF), jnp.float32),
        pltpu.SemaphoreType.DMA,
        pltpu.VMEM_SHARED((NPAD, HALF), jnp.float32),
    ],
)


def _prep_body(x_ref, eb_ref, xe0_ref, xe1_ref):
    xe0_ref[...] = x_ref[:, :HALF] + eb_ref[0:1, :HALF]
    xe1_ref[...] = x_ref[:, HALF:] + eb_ref[0:1, HALF:]


_prep = pl.pallas_call(
    _prep_body,
    grid=(NN // RB,),
    in_specs=[
        pl.BlockSpec((RB, DIM), lambda i: (i, 0)),
        pl.BlockSpec((1, DIM), lambda i: (0, 0)),
    ],
    out_specs=[pl.BlockSpec((RB, HALF), lambda i: (i, 0))] * 2,
    out_shape=[jax.ShapeDtypeStruct((NN, HALF), jnp.float32)] * 2,
)


def _mlp1_body(fac_ref, x_ref, a0_ref, a1_ref, w1_ref, b1_ref, h1_ref, st_ref):
    i = pl.program_id(0)
    aggr = jnp.concatenate([a0_ref[...], a1_ref[...]], axis=1)
    h0 = fac_ref[0] * x_ref[...] + aggr
    h1 = jnp.dot(h0, w1_ref[...], preferred_element_type=jnp.float32) + b1_ref[...]
    h1_ref[...] = h1
    s1 = jnp.sum(h1, axis=0, keepdims=True)
    s2 = jnp.sum(h1 * h1, axis=0, keepdims=True)
    upd = jnp.concatenate([s1, s2], axis=0)

    @pl.when(i == 0)
    def _():
        st_ref[...] = upd

    @pl.when(i != 0)
    def _():
        st_ref[...] += upd


_mlp1 = pl.pallas_call(
    _mlp1_body,
    grid=(NN // RB,),
    in_specs=[
        pl.BlockSpec(memory_space=pltpu.SMEM),
        pl.BlockSpec((RB, DIM), lambda i: (i, 0)),
        pl.BlockSpec((RB, HALF), lambda i: (i, 0)),
        pl.BlockSpec((RB, HALF), lambda i: (i, 0)),
        pl.BlockSpec((DIM, DIM), lambda i: (0, 0)),
        pl.BlockSpec((1, DIM), lambda i: (0, 0)),
    ],
    out_specs=[
        pl.BlockSpec((RB, DIM), lambda i: (i, 0)),
        pl.BlockSpec((2, DIM), lambda i: (0, 0)),
    ],
    out_shape=[
        jax.ShapeDtypeStruct((NN, DIM), jnp.float32),
        jax.ShapeDtypeStruct((2, DIM), jnp.float32),
    ],
)


def _make_mlp2(emit_xe):
    def body(*refs):
        if emit_xe:
            (h1_ref, st_ref, gb_ref, w2_ref, b2_ref, ebn_ref,
             xn_ref, xe0_ref, xe1_ref) = refs
        else:
            h1_ref, st_ref, gb_ref, w2_ref, b2_ref, xn_ref = refs
        mu = st_ref[0:1, :] * (1.0 / NN)
        ex2 = st_ref[1:2, :] * (1.0 / NN)
        var = ex2 - mu * mu
        scale = lax.rsqrt(var + 1e-5) * gb_ref[0:1, :]
        shift = gb_ref[1:2, :] - mu * scale
        h2 = jnp.maximum(h1_ref[...] * scale + shift, 0.0)
        xn = jnp.dot(h2, w2_ref[...], preferred_element_type=jnp.float32)
        xn = jnp.maximum(xn + b2_ref[...], 0.0)
        xn_ref[...] = xn
        if emit_xe:
            xe0_ref[...] = xn[:, :HALF] + ebn_ref[0:1, :HALF]
            xe1_ref[...] = xn[:, HALF:] + ebn_ref[0:1, HALF:]

    in_specs = [
        pl.BlockSpec((RB, DIM), lambda i: (i, 0)),
        pl.BlockSpec((2, DIM), lambda i: (0, 0)),
        pl.BlockSpec((2, DIM), lambda i: (0, 0)),
        pl.BlockSpec((DIM, DIM), lambda i: (0, 0)),
        pl.BlockSpec((1, DIM), lambda i: (0, 0)),
    ]
    out_specs = [pl.BlockSpec((RB, DIM), lambda i: (i, 0))]
    out_shape = [jax.ShapeDtypeStruct((NN, DIM), jnp.float32)]
    if emit_xe:
        in_specs.append(pl.BlockSpec((1, DIM), lambda i: (0, 0)))
        out_specs += [pl.BlockSpec((RB, HALF), lambda i: (i, 0))] * 2
        out_shape += [jax.ShapeDtypeStruct((NN, HALF), jnp.float32)] * 2
    return pl.pallas_call(
        body,
        grid=(NN // RB,),
        in_specs=in_specs,
        out_specs=out_specs,
        out_shape=out_shape,
    )


_mlp2 = _make_mlp2(True)
_mlp2_last = _make_mlp2(False)


@jax.jit
def _run(x, src, dst, attr, params):
    order = jnp.argsort(src)
    src = src[order]
    dst = dst[order]
    attr = attr[order]

    def to_blocks(v, fill):
        vb = jnp.pad(v, (0, EPAD - NE), constant_values=fill)
        return vb.reshape(NSUB, NBLK, EB)

    src_b = to_blocks(src, 0)
    dst_b = to_blocks(dst, TRASH)
    attr_b = to_blocks(attr, 0.0)
    zrows = jnp.zeros((ZR, HALF), jnp.float32)

    xe0, xe1 = _prep(x, params[0]["eb"].reshape(1, DIM))
    h = x
    nlayers = len(params)
    for l, p in enumerate(params):
        w2 = p["eW"].reshape(2, HALF)
        a0, a1 = _sc_aggregate(xe0, xe1, src_b, dst_b, attr_b, w2, zrows)
        fac = (1.0 + p["eps"]).reshape(1)
        h1, st = _mlp1(fac, h, a0, a1, p["W1"], p["b1"].reshape(1, DIM))
        gb = jnp.stack([p["g"], p["be"]])
        b2 = p["b2"].reshape(1, DIM)
        if l + 1 < nlayers:
            ebn = params[l + 1]["eb"].reshape(1, DIM)
            h, xe0, xe1 = _mlp2(h1, st, gb, p["W2"], b2, ebn)
        else:
            (h,) = _mlp2_last(h1, st, gb, p["W2"], b2)
    return h


def kernel(x, edge_index, edge_attr, params):
    x = jnp.asarray(x, jnp.float32)
    src = edge_index[0].astype(jnp.int32)
    dst = edge_index[1].astype(jnp.int32)
    attr = edge_attr[:, 0].astype(jnp.float32)
    return _run(x, src, dst, attr, params)


# speedup vs baseline: 1.1666x; 1.1666x over previous
import functools

import jax
import jax.numpy as jnp
from jax import lax
from jax.experimental import pallas as pl
from jax.experimental.pallas import tpu as pltpu
from jax.experimental.pallas import tpu_sc as plsc

DIM = 256
HALF = 128
NN = 10000
NE = 160000
NSUB = 16
EB = 128
NBLK = 80
CH = 4
NBLK2 = 84
EPAD = NSUB * NBLK * EB
NPAD = 10112
TRASH = NN
ZR = NPAD // NSUB
RB = 2000


def _sc_body(xe0, xe1, src_b, dst_b, attr_b, w2, zrows,
             out0, out1,
             src_st, dst_st, attr_st, w_v, gbuf, sem_g, aggr_sh):
    c = lax.axis_index("c")
    s = lax.axis_index("s")

    pltpu.sync_copy(src_b.at[s], src_st)
    pltpu.sync_copy(dst_b.at[s], dst_st)
    pltpu.sync_copy(attr_b.at[s], attr_st)
    pltpu.sync_copy(w2, w_v)

    pltpu.sync_copy(zrows, aggr_sh.at[pl.ds(s * ZR, ZR)])
    plsc.subcore_barrier()

    wv = [w_v[c, pl.ds(16 * f, 16)] for f in range(8)]

    def block_body(blk, carry):
        @pl.when(c == 0)
        def _():
            pltpu.async_copy(xe0.at[src_st.at[blk]], gbuf, sem_g).wait()

        @pl.when(c == 1)
        def _():
            pltpu.async_copy(xe1.at[src_st.at[blk]], gbuf, sem_g).wait()

        def grp_body(eg, carry2):
            a16 = attr_st[blk, pl.ds(eg * 16, 16)]
            for j in range(16):
                a = a16[j]
                e = eg * 16 + j
                for f in range(8):
                    v = gbuf[e, pl.ds(16 * f, 16)]
                    gbuf[e, pl.ds(16 * f, 16)] = jnp.maximum(v + a * wv[f], 0.0)
            return carry2

        lax.fori_loop(0, EB // 16, grp_body, 0)

        pltpu.sync_copy(gbuf, aggr_sh.at[dst_st.at[blk]], add=True)
        return carry

    lax.fori_loop(0, NBLK, block_body, 0)
    plsc.subcore_barrier()

    @pl.when(c == 0)
    def _():
        pltpu.sync_copy(aggr_sh.at[pl.ds(s * ZR, ZR)], out0.at[pl.ds(s * ZR, ZR)])

    @pl.when(c == 1)
    def _():
        pltpu.sync_copy(aggr_sh.at[pl.ds(s * ZR, ZR)], out1.at[pl.ds(s * ZR, ZR)])


_sc_aggregate = pl.kernel(
    _sc_body,
    out_type=(jax.ShapeDtypeStruct((NPAD, HALF), jnp.float32),
              jax.ShapeDtypeStruct((NPAD, HALF), jnp.float32)),
    mesh=plsc.VectorSubcoreMesh(core_axis_name="c", subcore_axis_name="s"),
    scratch_types=[
        pltpu.VMEM((NBLK, EB), jnp.int32),
        pltpu.VMEM((NBLK, EB), jnp.int32),
        pltpu.VMEM((NBLK, EB), jnp.float32),
        pltpu.VMEM((2, HALF), jnp.float32),
        pltpu.VMEM((EB, HALF), jnp.float32),
        pltpu.SemaphoreType.DMA,
        pltpu.VMEM_SHARED((NPAD, HALF), jnp.float32),
    ],
)


def _prep_body(x_ref, eb_ref, xe0_ref, xe1_ref):
    xe0_ref[...] = x_ref[:, :HALF] + eb_ref[0:1, :HALF]
    xe1_ref[...] = x_ref[:, HALF:] + eb_ref[0:1, HALF:]


_prep = pl.pallas_call(
    _prep_body,
    grid=(NN // RB,),
    in_specs=[
        pl.BlockSpec((RB, DIM), lambda i: (i, 0)),
        pl.BlockSpec((1, DIM), lambda i: (0, 0)),
    ],
    out_specs=[pl.BlockSpec((RB, HALF), lambda i: (i, 0))] * 2,
    out_shape=[jax.ShapeDtypeStruct((NN, HALF), jnp.float32)] * 2,
)


def _mlp1_body(fac_ref, x_ref, a0_ref, a1_ref, w1_ref, b1_ref, h1_ref, st_ref):
    i = pl.program_id(0)
    aggr = jnp.concatenate([a0_ref[...], a1_ref[...]], axis=1)
    h0 = fac_ref[0] * x_ref[...] + aggr
    h1 = jnp.dot(h0, w1_ref[...], preferred_element_type=jnp.float32) + b1_ref[...]
    h1_ref[...] = h1
    s1 = jnp.sum(h1, axis=0, keepdims=True)
    s2 = jnp.sum(h1 * h1, axis=0, keepdims=True)
    upd = jnp.concatenate([s1, s2], axis=0)

    @pl.when(i == 0)
    def _():
        st_ref[...] = upd

    @pl.when(i != 0)
    def _():
        st_ref[...] += upd


_mlp1 = pl.pallas_call(
    _mlp1_body,
    grid=(NN // RB,),
    in_specs=[
        pl.BlockSpec(memory_space=pltpu.SMEM),
        pl.BlockSpec((RB, DIM), lambda i: (i, 0)),
        pl.BlockSpec((RB, HALF), lambda i: (i, 0)),
        pl.BlockSpec((RB, HALF), lambda i: (i, 0)),
        pl.BlockSpec((DIM, DIM), lambda i: (0, 0)),
        pl.BlockSpec((1, DIM), lambda i: (0, 0)),
    ],
    out_specs=[
        pl.BlockSpec((RB, DIM), lambda i: (i, 0)),
        pl.BlockSpec((2, DIM), lambda i: (0, 0)),
    ],
    out_shape=[
        jax.ShapeDtypeStruct((NN, DIM), jnp.float32),
        jax.ShapeDtypeStruct((2, DIM), jnp.float32),
    ],
)


def _make_mlp2(emit_xe):
    def body(*refs):
        if emit_xe:
            (h1_ref, st_ref, gb_ref, w2_ref, b2_ref, ebn_ref,
             xn_ref, xe0_ref, xe1_ref) = refs
        else:
            h1_ref, st_ref, gb_ref, w2_ref, b2_ref, xn_ref = refs
        mu = st_ref[0:1, :] * (1.0 / NN)
        ex2 = st_ref[1:2, :] * (1.0 / NN)
        var = ex2 - mu * mu
        scale = lax.rsqrt(var + 1e-5) * gb_ref[0:1, :]
        shift = gb_ref[1:2, :] - mu * scale
        h2 = jnp.maximum(h1_ref[...] * scale + shift, 0.0)
        xn = jnp.dot(h2, w2_ref[...], preferred_element_type=jnp.float32)
        xn = jnp.maximum(xn + b2_ref[...], 0.0)
        xn_ref[...] = xn
        if emit_xe:
            xe0_ref[...] = xn[:, :HALF] + ebn_ref[0:1, :HALF]
            xe1_ref[...] = xn[:, HALF:] + ebn_ref[0:1, HALF:]

    in_specs = [
        pl.BlockSpec((RB, DIM), lambda i: (i, 0)),
        pl.BlockSpec((2, DIM), lambda i: (0, 0)),
        pl.BlockSpec((2, DIM), lambda i: (0, 0)),
        pl.BlockSpec((DIM, DIM), lambda i: (0, 0)),
        pl.BlockSpec((1, DIM), lambda i: (0, 0)),
    ]
    out_specs = [pl.BlockSpec((RB, DIM), lambda i: (i, 0))]
    out_shape = [jax.ShapeDtypeStruct((NN, DIM), jnp.float32)]
    if emit_xe:
        in_specs.append(pl.BlockSpec((1, DIM), lambda i: (0, 0)))
        out_specs += [pl.BlockSpec((RB, HALF), lambda i: (i, 0))] * 2
        out_shape += [jax.ShapeDtypeStruct((NN, HALF), jnp.float32)] * 2
    return pl.pallas_call(
        body,
        grid=(NN // RB,),
        in_specs=in_specs,
        out_specs=out_specs,
        out_shape=out_shape,
    )


_mlp2 = _make_mlp2(True)
_mlp2_last = _make_mlp2(False)


@jax.jit
def _run(x, src, dst, attr, params):
    def to_blocks(v, fill):
        vb = jnp.pad(v, (0, EPAD - NE), constant_values=fill)
        return vb.reshape(NSUB, NBLK, EB)

    src_b = to_blocks(src, 0)
    dst_b = to_blocks(dst, TRASH)
    attr_b = to_blocks(attr, 0.0)
    zrows = jnp.zeros((ZR, HALF), jnp.float32)

    xe0, xe1 = _prep(x, params[0]["eb"].reshape(1, DIM))
    h = x
    nlayers = len(params)
    for l, p in enumerate(params):
        w2 = p["eW"].reshape(2, HALF)
        a0, a1 = _sc_aggregate(xe0, xe1, src_b, dst_b, attr_b, w2, zrows)
        fac = (1.0 + p["eps"]).reshape(1)
        h1, st = _mlp1(fac, h, a0, a1, p["W1"], p["b1"].reshape(1, DIM))
        gb = jnp.stack([p["g"], p["be"]])
        b2 = p["b2"].reshape(1, DIM)
        if l + 1 < nlayers:
            ebn = params[l + 1]["eb"].reshape(1, DIM)
            h, xe0, xe1 = _mlp2(h1, st, gb, p["W2"], b2, ebn)
        else:
            (h,) = _mlp2_last(h1, st, gb, p["W2"], b2)
    return h


def kernel(x, edge_index, edge_attr, params):
    x = jnp.asarray(x, jnp.float32)
    src = edge_index[0].astype(jnp.int32)
    dst = edge_index[1].astype(jnp.int32)
    attr = edge_attr[:, 0].astype(jnp.float32)
    return _run(x, src, dst, attr, params)


# speedup vs baseline: 1.5507x; 1.3293x over previous
import functools

import jax
import jax.numpy as jnp
from jax import lax
from jax.experimental import pallas as pl
from jax.experimental.pallas import tpu as pltpu
from jax.experimental.pallas import tpu_sc as plsc

DIM = 256
HALF = 128
NN = 10000
NE = 160000
NSUB = 16
EB = 128
NBLK = 79
EPAD = NSUB * NBLK * EB
NPAD = 10112
TRASH = NN
ZR = NPAD // NSUB
RB = 2000


def _sc_body(xe0, xe1, src_b, dst_b, attr_b, w2, zrows,
             out0, out1,
             src_st, dst_st, attr_st, w_v, gbuf, sem_g, aggr_sh):
    c = lax.axis_index("c")
    s = lax.axis_index("s")

    pltpu.sync_copy(src_b.at[s], src_st)
    pltpu.sync_copy(dst_b.at[s], dst_st)
    pltpu.sync_copy(attr_b.at[s], attr_st)
    pltpu.sync_copy(w2, w_v)

    pltpu.sync_copy(zrows, aggr_sh.at[pl.ds(s * ZR, ZR)])
    plsc.subcore_barrier()

    wv = [w_v[c, pl.ds(16 * f, 16)] for f in range(8)]

    def block_body(blk, carry):
        @pl.when(c == 0)
        def _():
            pltpu.async_copy(xe0.at[src_st.at[blk]], gbuf, sem_g).wait()

        @pl.when(c == 1)
        def _():
            pltpu.async_copy(xe1.at[src_st.at[blk]], gbuf, sem_g).wait()

        def grp_body(eg, carry2):
            a16 = attr_st[blk, pl.ds(eg * 16, 16)]
            for j in range(16):
                a = a16[j]
                e = eg * 16 + j
                for f in range(8):
                    v = gbuf[e, pl.ds(16 * f, 16)]
                    gbuf[e, pl.ds(16 * f, 16)] = jnp.maximum(v + a * wv[f], 0.0)
            return carry2

        lax.fori_loop(0, EB // 16, grp_body, 0)

        pltpu.sync_copy(gbuf, aggr_sh.at[dst_st.at[blk]], add=True)
        return carry

    lax.fori_loop(0, NBLK, block_body, 0)
    plsc.subcore_barrier()

    @pl.when(c == 0)
    def _():
        pltpu.sync_copy(aggr_sh.at[pl.ds(s * ZR, ZR)], out0.at[pl.ds(s * ZR, ZR)])

    @pl.when(c == 1)
    def _():
        pltpu.sync_copy(aggr_sh.at[pl.ds(s * ZR, ZR)], out1.at[pl.ds(s * ZR, ZR)])


_sc_aggregate = pl.kernel(
    _sc_body,
    out_type=(jax.ShapeDtypeStruct((NPAD, HALF), jnp.float32),
              jax.ShapeDtypeStruct((NPAD, HALF), jnp.float32)),
    mesh=plsc.VectorSubcoreMesh(core_axis_name="c", subcore_axis_name="s"),
    scratch_types=[
        pltpu.VMEM((NBLK, EB), jnp.int32),
        pltpu.VMEM((NBLK, EB), jnp.int32),
        pltpu.VMEM((NBLK, EB), jnp.float32),
        pltpu.VMEM((2, HALF), jnp.float32),
        pltpu.VMEM((EB, HALF), jnp.float32),
        pltpu.SemaphoreType.DMA,
        pltpu.VMEM_SHARED((NPAD, HALF), jnp.float32),
    ],
)


def _prep_body(x_ref, eb_ref, xe0_ref, xe1_ref):
    xe0_ref[...] = x_ref[:, :HALF] + eb_ref[0:1, :HALF]
    xe1_ref[...] = x_ref[:, HALF:] + eb_ref[0:1, HALF:]


_prep = pl.pallas_call(
    _prep_body,
    grid=(NN // RB,),
    in_specs=[
        pl.BlockSpec((RB, DIM), lambda i: (i, 0)),
        pl.BlockSpec((1, DIM), lambda i: (0, 0)),
    ],
    out_specs=[pl.BlockSpec((RB, HALF), lambda i: (i, 0))] * 2,
    out_shape=[jax.ShapeDtypeStruct((NN, HALF), jnp.float32)] * 2,
)


def _mlp1_body(fac_ref, x_ref, a0_ref, a1_ref, w1_ref, b1_ref, h1_ref, st_ref):
    i = pl.program_id(0)
    aggr = jnp.concatenate([a0_ref[...], a1_ref[...]], axis=1)
    h0 = fac_ref[0] * x_ref[...] + aggr
    h1 = jnp.dot(h0, w1_ref[...], preferred_element_type=jnp.float32) + b1_ref[...]
    h1_ref[...] = h1
    s1 = jnp.sum(h1, axis=0, keepdims=True)
    s2 = jnp.sum(h1 * h1, axis=0, keepdims=True)
    upd = jnp.concatenate([s1, s2], axis=0)

    @pl.when(i == 0)
    def _():
        st_ref[...] = upd

    @pl.when(i != 0)
    def _():
        st_ref[...] += upd


_mlp1 = pl.pallas_call(
    _mlp1_body,
    grid=(NN // RB,),
    in_specs=[
        pl.BlockSpec(memory_space=pltpu.SMEM),
        pl.BlockSpec((RB, DIM), lambda i: (i, 0)),
        pl.BlockSpec((RB, HALF), lambda i: (i, 0)),
        pl.BlockSpec((RB, HALF), lambda i: (i, 0)),
        pl.BlockSpec((DIM, DIM), lambda i: (0, 0)),
        pl.BlockSpec((1, DIM), lambda i: (0, 0)),
    ],
    out_specs=[
        pl.BlockSpec((RB, DIM), lambda i: (i, 0)),
        pl.BlockSpec((2, DIM), lambda i: (0, 0)),
    ],
    out_shape=[
        jax.ShapeDtypeStruct((NN, DIM), jnp.float32),
        jax.ShapeDtypeStruct((2, DIM), jnp.float32),
    ],
)


def _make_mlp2(emit_xe):
    def body(*refs):
        if emit_xe:
            (h1_ref, st_ref, gb_ref, w2_ref, b2_ref, ebn_ref,
             xn_ref, xe0_ref, xe1_ref) = refs
        else:
            h1_ref, st_ref, gb_ref, w2_ref, b2_ref, xn_ref = refs
        mu = st_ref[0:1, :] * (1.0 / NN)
        ex2 = st_ref[1:2, :] * (1.0 / NN)
        var = ex2 - mu * mu
        scale = lax.rsqrt(var + 1e-5) * gb_ref[0:1, :]
        shift = gb_ref[1:2, :] - mu * scale
        h2 = jnp.maximum(h1_ref[...] * scale + shift, 0.0)
        xn = jnp.dot(h2, w2_ref[...], preferred_element_type=jnp.float32)
        xn = jnp.maximum(xn + b2_ref[...], 0.0)
        xn_ref[...] = xn
        if emit_xe:
            xe0_ref[...] = xn[:, :HALF] + ebn_ref[0:1, :HALF]
            xe1_ref[...] = xn[:, HALF:] + ebn_ref[0:1, HALF:]

    in_specs = [
        pl.BlockSpec((RB, DIM), lambda i: (i, 0)),
        pl.BlockSpec((2, DIM), lambda i: (0, 0)),
        pl.BlockSpec((2, DIM), lambda i: (0, 0)),
        pl.BlockSpec((DIM, DIM), lambda i: (0, 0)),
        pl.BlockSpec((1, DIM), lambda i: (0, 0)),
    ]
    out_specs = [pl.BlockSpec((RB, DIM), lambda i: (i, 0))]
    out_shape = [jax.ShapeDtypeStruct((NN, DIM), jnp.float32)]
    if emit_xe:
        in_specs.append(pl.BlockSpec((1, DIM), lambda i: (0, 0)))
        out_specs += [pl.BlockSpec((RB, HALF), lambda i: (i, 0))] * 2
        out_shape += [jax.ShapeDtypeStruct((NN, HALF), jnp.float32)] * 2
    return pl.pallas_call(
        body,
        grid=(NN // RB,),
        in_specs=in_specs,
        out_specs=out_specs,
        out_shape=out_shape,
    )


_mlp2 = _make_mlp2(True)
_mlp2_last = _make_mlp2(False)


@jax.jit
def _run(x, src, dst, attr, params):
    def to_blocks(v, fill):
        vb = jnp.pad(v, (0, EPAD - NE), constant_values=fill)
        return vb.reshape(NSUB, NBLK, EB)

    src_b = to_blocks(src, 0)
    dst_b = to_blocks(dst, TRASH)
    attr_b = to_blocks(attr, 0.0)
    zrows = jnp.zeros((ZR, HALF), jnp.float32)

    xe0, xe1 = _prep(x, params[0]["eb"].reshape(1, DIM))
    h = x
    nlayers = len(params)
    for l, p in enumerate(params):
        w2 = p["eW"].reshape(2, HALF)
        a0, a1 = _sc_aggregate(xe0, xe1, src_b, dst_b, attr_b, w2, zrows)
        fac = (1.0 + p["eps"]).reshape(1)
        h1, st = _mlp1(fac, h, a0, a1, p["W1"], p["b1"].reshape(1, DIM))
        gb = jnp.stack([p["g"], p["be"]])
        b2 = p["b2"].reshape(1, DIM)
        if l + 1 < nlayers:
            ebn = params[l + 1]["eb"].reshape(1, DIM)
            h, xe0, xe1 = _mlp2(h1, st, gb, p["W2"], b2, ebn)
        else:
            (h,) = _mlp2_last(h1, st, gb, p["W2"], b2)
    return h


def kernel(x, edge_index, edge_attr, params):
    x = jnp.asarray(x, jnp.float32)
    src = edge_index[0].astype(jnp.int32)
    dst = edge_index[1].astype(jnp.int32)
    attr = edge_attr[:, 0].astype(jnp.float32)
    return _run(x, src, dst, attr, params)
